# baseline (speedup 1.0000x reference)
import functools

import jax
import jax.numpy as jnp
from jax import lax
from jax.experimental import pallas as pl
from jax.experimental.pallas import tpu as pltpu
from jax.experimental.pallas import tpu_sc as plsc

N_LAYERS = 8
D = 2048
ROW_BLOCK = 1024
BLOCKS_PER_LAYER = D // ROW_BLOCK
LANES = 16
CHUNKS = D // LANES
UNROLL = 8
MAX_FINITE_BITS = 0x7F7FFFFF


def _scores_body(w_ref, o_ref):
    b = pl.program_id(1)

    @pl.when(b == 0)
    def _():
        o_ref[...] = jnp.zeros_like(o_ref)

    w = w_ref[0]
    o_ref[...] += jnp.sum(w * w, axis=0)[None, None, :]


_scores_call = pl.pallas_call(
    _scores_body,
    grid=(N_LAYERS, BLOCKS_PER_LAYER),
    in_specs=[pl.BlockSpec((1, ROW_BLOCK, D), lambda i, b: (i, b, 0))],
    out_specs=pl.BlockSpec((1, 1, D), lambda i, b: (i, 0, 0)),
    out_shape=jax.ShapeDtypeStruct((N_LAYERS, 1, D), jnp.float32),
)


@functools.cache
def _build_select_call():
    return functools.partial(
        pl.kernel,
        mesh=plsc.VectorSubcoreMesh(core_axis_name="c", subcore_axis_name="s"),
        out_type=jax.ShapeDtypeStruct((LANES,), jnp.float32),
        scratch_types=[
            pltpu.VMEM((D,), jnp.float32),
            pltpu.VMEM((LANES,), jnp.float32),
            pltpu.VMEM((LANES,), jnp.float32),
            pltpu.VMEM((LANES,), jnp.float32),
            pltpu.VMEM((N_LAYERS * LANES,), jnp.float32),
            pltpu.VMEM((LANES,), jnp.float32),
            pltpu.VMEM_SHARED((N_LAYERS * LANES,), jnp.float32),
            pltpu.VMEM((LANES,), jnp.int32),
        ],
        compiler_params=pltpu.CompilerParams(needs_layout_passes=False),
    )(_select_body)


def _select_body(scores_hbm, s_hbm, y_hbm, out_hbm,
                 scores_v, s_v, y_v, row_v, prod_v, out_v, shared, hist_ref):
    c = lax.axis_index("c")
    sid = lax.axis_index("s")
    lane = lax.iota(jnp.int32, LANES)

    @pl.when((c == 0) & (sid < N_LAYERS))
    def _():
        layer = sid
        pltpu.sync_copy(scores_hbm.at[layer], scores_v)
        pltpu.sync_copy(s_hbm, s_v)
        pltpu.sync_copy(y_hbm, y_v)
        mask = lane == layer
        s_vec = s_v[...]
        y_vec = y_v[...]
        t_vec = s_vec.astype(jnp.int32)
        k_vec = t_vec + jnp.where(t_vec.astype(jnp.float32) < s_vec, 1, 0)
        k_vec = jnp.minimum(jnp.maximum(k_vec, 0), D)
        k = jnp.sum(jnp.where(mask, k_vec, 0))
        y_i = jnp.sum(jnp.where(mask, y_vec, 0.0))

        def mmbody(jj, mnmx):
            mn, mx = mnmx
            for u in range(UNROLL):
                v = plsc.bitcast(
                    scores_v[pl.ds((jj * UNROLL + u) * LANES, LANES)],
                    jnp.int32)
                mn = jnp.minimum(mn, v)
                mx = jnp.maximum(mx, v)
            return (mn, mx)

        mn_v, mx_v = lax.fori_loop(
            0, CHUNKS // UNROLL, mmbody,
            (jnp.full((LANES,), MAX_FINITE_BITS, jnp.int32),
             jnp.zeros((LANES,), jnp.int32)))

        k_splat = jnp.full((LANES,), k, dtype=jnp.int32)
        ones_v = jnp.full((LANES,), 1, dtype=jnp.int32)

        def rcond(st):
            lo_v, hi_v, _ = st
            return jnp.max(hi_v - lo_v) > 0

        def rbody(st):
            lo_v, hi_v, below_v = st
            width_v = hi_v - lo_v + 1
            e_v = (plsc.bitcast(width_v.astype(jnp.float32), jnp.int32)
                   >> 23) - 127
            sh_v = jnp.maximum(e_v - 3, 0)
            hist_ref[...] = jnp.zeros((LANES,), jnp.int32)

            def hbody(jj, carry):
                for u in range(UNROLL):
                    v = plsc.bitcast(
                        scores_v[pl.ds((jj * UNROLL + u) * LANES, LANES)],
                        jnp.int32)
                    inr = (v >= lo_v) & (v <= hi_v)
                    b = (v - lo_v) >> sh_v
                    plsc.addupdate_scatter(hist_ref, [b], ones_v, mask=inr)
                return carry

            lax.fori_loop(0, CHUNKS // UNROLL, hbody, 0)
            hist = hist_ref[...]
            cum = plsc.cumsum(hist)
            cond = (below_v + cum) >= k_splat
            b_v = plsc.all_reduce_ffs(cond)
            below2 = below_v + jnp.full(
                (LANES,), jnp.sum(jnp.where(lane < b_v, hist, 0)), jnp.int32)
            lo2 = lo_v + (b_v << sh_v)
            hi2 = jnp.minimum(lo_v + ((b_v + 1) << sh_v) - 1, hi_v)
            return (lo2, hi2, below2)

        t_bits, _, _ = lax.while_loop(
            rcond, rbody,
            (jnp.full((LANES,), jnp.min(mn_v), jnp.int32),
             jnp.full((LANES,), jnp.max(mx_v), jnp.int32),
             jnp.zeros((LANES,), jnp.int32)))

        def fbody(jj, carry):
            sm, cl = carry
            for u in range(UNROLL):
                sv = scores_v[pl.ds((jj * UNROLL + u) * LANES, LANES)]
                bv = plsc.bitcast(sv, jnp.int32)
                lt = bv < t_bits
                sm = sm + jnp.where(lt, sv, 0.0)
                cl = cl + jnp.where(lt, 1, 0)
            return (sm, cl)

        sm, cl = lax.fori_loop(
            0, CHUNKS // UNROLL, fbody,
            (jnp.zeros((LANES,), jnp.float32), jnp.zeros((LANES,), jnp.int32)))
        sum_lt = jnp.sum(sm)
        cnt_lt = jnp.sum(cl)
        rem_vec = jnp.full((LANES,), k - cnt_lt, dtype=jnp.int32)
        val_vec = (jnp.full((LANES,), sum_lt, dtype=jnp.float32)
                   + rem_vec.astype(jnp.float32)
                   * plsc.bitcast(t_bits, jnp.float32))
        row_v[...] = jnp.where(mask, y_vec * val_vec, 0.0)
        pltpu.sync_copy(row_v, shared.at[pl.ds(layer * LANES, LANES)])

    plsc.subcore_barrier()

    @pl.when((c == 0) & (sid == 0))
    def _():
        pltpu.sync_copy(shared, prod_v)

        def abody(j, acc):
            return acc + prod_v[pl.ds(j * LANES, LANES)]

        acc = lax.fori_loop(0, N_LAYERS, abody,
                            jnp.zeros((LANES,), jnp.float32))
        out_v[...] = jnp.full((LANES,), jnp.sum(acc), dtype=jnp.float32)
        pltpu.sync_copy(out_v, out_hbm)


def kernel(weights, s, y):
    scores = _scores_call(weights).reshape(N_LAYERS, D)
    s_pad = jnp.zeros((LANES,), jnp.float32).at[:N_LAYERS].set(s)
    y_pad = jnp.zeros((LANES,), jnp.float32).at[:N_LAYERS].set(y)
    out16 = _build_select_call()(scores, s_pad, y_pad)
    return out16[0]


# speedup vs baseline: 1.0526x; 1.0526x over previous
import functools

import jax
import jax.numpy as jnp
from jax import lax
from jax.experimental import pallas as pl
from jax.experimental.pallas import tpu as pltpu
from jax.experimental.pallas import tpu_sc as plsc

N_LAYERS = 8
D = 2048
ROW_BLOCK = 1024
BLOCKS_PER_LAYER = D // ROW_BLOCK
LANES = 16
CHUNKS = D // LANES
UNROLL = 8
MAX_FINITE_BITS = 0x7F7FFFFF


def _scores_body(w_ref, o_ref):
    b = pl.program_id(1)

    @pl.when(b == 0)
    def _():
        o_ref[...] = jnp.zeros_like(o_ref)

    w = w_ref[0]
    o_ref[...] += jnp.sum(w * w, axis=0)[None, None, :]


_scores_call = pl.pallas_call(
    _scores_body,
    grid=(N_LAYERS, BLOCKS_PER_LAYER),
    in_specs=[pl.BlockSpec((1, ROW_BLOCK, D), lambda i, b: (i, b, 0))],
    out_specs=pl.BlockSpec((1, 1, D), lambda i, b: (i, 0, 0)),
    out_shape=jax.ShapeDtypeStruct((N_LAYERS, 1, D), jnp.float32),
)


@functools.cache
def _build_select_call():
    return functools.partial(
        pl.kernel,
        mesh=plsc.VectorSubcoreMesh(core_axis_name="c", subcore_axis_name="s"),
        out_type=jax.ShapeDtypeStruct((LANES,), jnp.float32),
        scratch_types=[
            pltpu.VMEM((D,), jnp.float32),
            pltpu.VMEM((LANES,), jnp.float32),
            pltpu.VMEM((LANES,), jnp.float32),
            pltpu.VMEM((LANES,), jnp.float32),
            pltpu.VMEM((N_LAYERS * LANES,), jnp.float32),
            pltpu.VMEM((LANES,), jnp.float32),
            pltpu.VMEM_SHARED((N_LAYERS * LANES,), jnp.float32),
        ],
        compiler_params=pltpu.CompilerParams(needs_layout_passes=False),
    )(_select_body)


def _select_body(scores_hbm, s_hbm, y_hbm, out_hbm,
                 scores_v, s_v, y_v, row_v, prod_v, out_v, shared):
    c = lax.axis_index("c")
    sid = lax.axis_index("s")
    lane = lax.iota(jnp.int32, LANES)

    @pl.when((c == 0) & (sid < N_LAYERS))
    def _():
        layer = sid
        pltpu.sync_copy(scores_hbm.at[layer], scores_v)
        pltpu.sync_copy(s_hbm, s_v)
        pltpu.sync_copy(y_hbm, y_v)
        mask = lane == layer
        s_vec = s_v[...]
        y_vec = y_v[...]
        t_vec = s_vec.astype(jnp.int32)
        k_vec = t_vec + jnp.where(t_vec.astype(jnp.float32) < s_vec, 1, 0)
        k_vec = jnp.minimum(jnp.maximum(k_vec, 0), D)
        k = jnp.sum(jnp.where(mask, k_vec, 0))
        y_i = jnp.sum(jnp.where(mask, y_vec, 0.0))

        def mmbody(jj, mnmx):
            mn, mx = mnmx
            for u in range(UNROLL):
                v = plsc.bitcast(
                    scores_v[pl.ds((jj * UNROLL + u) * LANES, LANES)],
                    jnp.int32)
                mn = jnp.minimum(mn, v)
                mx = jnp.maximum(mx, v)
            return (mn, mx)

        mn_v, mx_v = lax.fori_loop(
            0, CHUNKS // UNROLL, mmbody,
            (jnp.full((LANES,), MAX_FINITE_BITS, jnp.int32),
             jnp.zeros((LANES,), jnp.int32)))

        def count3(m1, m2, m3):
            m1v = jnp.full((LANES,), m1, dtype=jnp.int32)
            m2v = jnp.full((LANES,), m2, dtype=jnp.int32)
            m3v = jnp.full((LANES,), m3, dtype=jnp.int32)

            def cbody(jj, cs):
                c1, c2, c3 = cs
                for u in range(UNROLL):
                    v = plsc.bitcast(
                        scores_v[pl.ds((jj * UNROLL + u) * LANES, LANES)],
                        jnp.int32)
                    c1 = c1 + jnp.where(v <= m1v, 1, 0)
                    c2 = c2 + jnp.where(v <= m2v, 1, 0)
                    c3 = c3 + jnp.where(v <= m3v, 1, 0)
                return (c1, c2, c3)

            z = jnp.zeros((LANES,), jnp.int32)
            c1, c2, c3 = lax.fori_loop(0, CHUNKS // UNROLL, cbody, (z, z, z))
            return jnp.sum(c1), jnp.sum(c2), jnp.sum(c3)

        def bcond(lohi):
            lo, hi = lohi
            return lo < hi

        def bbody(lohi):
            lo, hi = lohi
            span = hi - lo
            q = span >> 2
            m1 = lo + q
            m2 = lo + (span >> 1)
            m3 = hi - q
            c1, c2, c3 = count3(m1, m2, m3)
            ge1 = c1 >= k
            ge2 = c2 >= k
            ge3 = c3 >= k
            lo2 = jnp.where(ge1, lo,
                            jnp.where(ge2, m1 + 1,
                                      jnp.where(ge3, m2 + 1, m3 + 1)))
            hi2 = jnp.where(ge1, m1,
                            jnp.where(ge2, m2,
                                      jnp.where(ge3, m3, hi)))
            return (lo2, hi2)

        lo, _ = lax.while_loop(
            bcond, bbody, (jnp.min(mn_v), jnp.max(mx_v)))
        t_bits = jnp.full((LANES,), lo, dtype=jnp.int32)

        def fbody(jj, carry):
            sm, cl = carry
            for u in range(UNROLL):
                sv = scores_v[pl.ds((jj * UNROLL + u) * LANES, LANES)]
                bv = plsc.bitcast(sv, jnp.int32)
                lt = bv < t_bits
                sm = sm + jnp.where(lt, sv, 0.0)
                cl = cl + jnp.where(lt, 1, 0)
            return (sm, cl)

        sm, cl = lax.fori_loop(
            0, CHUNKS // UNROLL, fbody,
            (jnp.zeros((LANES,), jnp.float32), jnp.zeros((LANES,), jnp.int32)))
        sum_lt = jnp.sum(sm)
        cnt_lt = jnp.sum(cl)
        rem_vec = jnp.full((LANES,), k - cnt_lt, dtype=jnp.int32)
        val_vec = (jnp.full((LANES,), sum_lt, dtype=jnp.float32)
                   + rem_vec.astype(jnp.float32)
                   * plsc.bitcast(t_bits, jnp.float32))
        row_v[...] = jnp.where(mask, y_vec * val_vec, 0.0)
        pltpu.sync_copy(row_v, shared.at[pl.ds(layer * LANES, LANES)])

    plsc.subcore_barrier()

    @pl.when((c == 0) & (sid == 0))
    def _():
        pltpu.sync_copy(shared, prod_v)

        def abody(j, acc):
            return acc + prod_v[pl.ds(j * LANES, LANES)]

        acc = lax.fori_loop(0, N_LAYERS, abody,
                            jnp.zeros((LANES,), jnp.float32))
        out_v[...] = jnp.full((LANES,), jnp.sum(acc), dtype=jnp.float32)
        pltpu.sync_copy(out_v, out_hbm)


def kernel(weights, s, y):
    scores = _scores_call(weights).reshape(N_LAYERS, D)
    s_pad = jnp.zeros((LANES,), jnp.float32).at[:N_LAYERS].set(s)
    y_pad = jnp.zeros((LANES,), jnp.float32).at[:N_LAYERS].set(y)
    out16 = _build_select_call()(scores, s_pad, y_pad)
    return out16[0]


# speedup vs baseline: 1.1033x; 1.0481x over previous
import functools

import jax
import jax.numpy as jnp
from jax import lax
from jax.experimental import pallas as pl
from jax.experimental.pallas import tpu as pltpu
from jax.experimental.pallas import tpu_sc as plsc

N_LAYERS = 8
D = 2048
ROW_BLOCK = 1024
BLOCKS_PER_LAYER = D // ROW_BLOCK
LANES = 16
CHUNKS = D // LANES
UNROLL = 8
MAX_FINITE_BITS = 0x7F7FFFFF


def _scores_body(w_ref, o_ref):
    b = pl.program_id(1)

    @pl.when(b == 0)
    def _():
        o_ref[...] = jnp.zeros_like(o_ref)

    w = w_ref[0]
    o_ref[...] += jnp.sum(w * w, axis=0)[None, None, :]


_scores_call = pl.pallas_call(
    _scores_body,
    grid=(N_LAYERS, BLOCKS_PER_LAYER),
    in_specs=[pl.BlockSpec((1, ROW_BLOCK, D), lambda i, b: (i, b, 0))],
    out_specs=pl.BlockSpec((1, 1, D), lambda i, b: (i, 0, 0)),
    out_shape=jax.ShapeDtypeStruct((N_LAYERS, 1, D), jnp.float32),
)


@functools.cache
def _build_select_call():
    return functools.partial(
        pl.kernel,
        mesh=plsc.VectorSubcoreMesh(core_axis_name="c", subcore_axis_name="s"),
        out_type=jax.ShapeDtypeStruct((LANES,), jnp.float32),
        scratch_types=[
            pltpu.VMEM((D,), jnp.float32),
            pltpu.VMEM((2 * LANES,), jnp.float32),
            pltpu.SemaphoreType.DMA,
            pltpu.SemaphoreType.DMA,
            pltpu.VMEM((LANES,), jnp.float32),
            pltpu.VMEM((N_LAYERS * LANES,), jnp.float32),
            pltpu.VMEM((LANES,), jnp.float32),
            pltpu.VMEM_SHARED((N_LAYERS * LANES,), jnp.float32),
        ],
        compiler_params=pltpu.CompilerParams(needs_layout_passes=False),
    )(_select_body)


def _select_body(scores_hbm, aux_hbm, out_hbm,
                 scores_v, aux_v, sem1, sem2, row_v, prod_v, out_v, shared):
    c = lax.axis_index("c")
    sid = lax.axis_index("s")
    lane = lax.iota(jnp.int32, LANES)

    @pl.when((c == 0) & (sid < N_LAYERS))
    def _():
        layer = sid
        h1 = pltpu.async_copy(scores_hbm.at[layer], scores_v, sem1)
        h2 = pltpu.async_copy(aux_hbm, aux_v, sem2)
        h2.wait()
        h1.wait()
        mask = lane == layer
        s_vec = aux_v[pl.ds(0, LANES)]
        y_vec = aux_v[pl.ds(LANES, LANES)]
        t_vec = s_vec.astype(jnp.int32)
        k_vec = t_vec + jnp.where(t_vec.astype(jnp.float32) < s_vec, 1, 0)
        k_vec = jnp.minimum(jnp.maximum(k_vec, 0), D)
        k = jnp.sum(jnp.where(mask, k_vec, 0))
        y_i = jnp.sum(jnp.where(mask, y_vec, 0.0))

        def mmbody(jj, mnmx):
            mn, mx = mnmx
            for u in range(UNROLL):
                v = plsc.bitcast(
                    scores_v[pl.ds((jj * UNROLL + u) * LANES, LANES)],
                    jnp.int32)
                mn = jnp.minimum(mn, v)
                mx = jnp.maximum(mx, v)
            return (mn, mx)

        mn_v, mx_v = lax.fori_loop(
            0, CHUNKS // UNROLL, mmbody,
            (jnp.full((LANES,), MAX_FINITE_BITS, jnp.int32),
             jnp.zeros((LANES,), jnp.int32)))

        def count_le(mid):
            mid_vec = jnp.full((LANES,), mid, dtype=jnp.int32)

            def cbody(jj, cnt):
                for u in range(UNROLL):
                    v = plsc.bitcast(
                        scores_v[pl.ds((jj * UNROLL + u) * LANES, LANES)],
                        jnp.int32)
                    cnt = cnt + jnp.where(v <= mid_vec, 1, 0)
                return cnt

            cnt = lax.fori_loop(0, CHUNKS // UNROLL, cbody,
                                jnp.zeros((LANES,), jnp.int32))
            return jnp.sum(cnt)

        def bcond(lohi):
            lo, hi = lohi
            return lo < hi

        def bbody(lohi):
            lo, hi = lohi
            mid = lo + ((hi - lo) >> 1)
            ge = count_le(mid) >= k
            return (jnp.where(ge, lo, mid + 1), jnp.where(ge, mid, hi))

        lo, _ = lax.while_loop(
            bcond, bbody, (jnp.min(mn_v), jnp.max(mx_v)))
        t_bits = jnp.full((LANES,), lo, dtype=jnp.int32)

        def fbody(jj, carry):
            sm, cl = carry
            for u in range(UNROLL):
                sv = scores_v[pl.ds((jj * UNROLL + u) * LANES, LANES)]
                bv = plsc.bitcast(sv, jnp.int32)
                lt = bv < t_bits
                sm = sm + jnp.where(lt, sv, 0.0)
                cl = cl + jnp.where(lt, 1, 0)
            return (sm, cl)

        sm, cl = lax.fori_loop(
            0, CHUNKS // UNROLL, fbody,
            (jnp.zeros((LANES,), jnp.float32), jnp.zeros((LANES,), jnp.int32)))
        sum_lt = jnp.sum(sm)
        cnt_lt = jnp.sum(cl)
        rem_vec = jnp.full((LANES,), k - cnt_lt, dtype=jnp.int32)
        val_vec = (jnp.full((LANES,), sum_lt, dtype=jnp.float32)
                   + rem_vec.astype(jnp.float32)
                   * plsc.bitcast(t_bits, jnp.float32))
        row_v[...] = jnp.where(mask, y_vec * val_vec, 0.0)
        pltpu.sync_copy(row_v, shared.at[pl.ds(layer * LANES, LANES)])

    plsc.subcore_barrier()

    @pl.when((c == 0) & (sid == 0))
    def _():
        pltpu.sync_copy(shared, prod_v)

        def abody(j, acc):
            return acc + prod_v[pl.ds(j * LANES, LANES)]

        acc = lax.fori_loop(0, N_LAYERS, abody,
                            jnp.zeros((LANES,), jnp.float32))
        out_v[...] = jnp.full((LANES,), jnp.sum(acc), dtype=jnp.float32)
        pltpu.sync_copy(out_v, out_hbm)


def kernel(weights, s, y):
    scores = _scores_call(weights).reshape(N_LAYERS, D)
    aux = jnp.zeros((2 * LANES,), jnp.float32)
    aux = aux.at[:N_LAYERS].set(s).at[LANES:LANES + N_LAYERS].set(y)
    out16 = _build_select_call()(scores, aux)
    return out16[0]


# speedup vs baseline: 1.1053x; 1.0018x over previous
import functools

import jax
import jax.numpy as jnp
from jax import lax
from jax.experimental import pallas as pl
from jax.experimental.pallas import tpu as pltpu
from jax.experimental.pallas import tpu_sc as plsc

N_LAYERS = 8
D = 2048
ROW_BLOCK = 1024
BLOCKS_PER_LAYER = D // ROW_BLOCK
LANES = 16
CHUNKS = D // LANES
UNROLL = 8
MAX_FINITE_BITS = 0x7F7FFFFF


def _scores_body(w1_ref, w2_ref, o_ref):
    b = pl.program_id(1)

    @pl.when(b == 0)
    def _():
        o_ref[...] = jnp.zeros_like(o_ref)

    w1 = w1_ref[0]
    w2 = w2_ref[0]
    o_ref[...] += (jnp.sum(w1 * w1, axis=0)
                   + jnp.sum(w2 * w2, axis=0))[None, None, :]


_scores_call = pl.pallas_call(
    _scores_body,
    grid=(N_LAYERS, BLOCKS_PER_LAYER),
    in_specs=[
        pl.BlockSpec((1, ROW_BLOCK // 2, D),
                     lambda i, b: (i, 2 * b, 0)),
        pl.BlockSpec((1, ROW_BLOCK // 2, D),
                     lambda i, b: (i, 2 * b + 1, 0)),
    ],
    out_specs=pl.BlockSpec((1, 1, D), lambda i, b: (i, 0, 0)),
    out_shape=jax.ShapeDtypeStruct((N_LAYERS, 1, D), jnp.float32),
)


@functools.cache
def _build_select_call():
    return functools.partial(
        pl.kernel,
        mesh=plsc.VectorSubcoreMesh(core_axis_name="c", subcore_axis_name="s"),
        out_type=jax.ShapeDtypeStruct((LANES,), jnp.float32),
        scratch_types=[
            pltpu.VMEM((D,), jnp.float32),
            pltpu.VMEM((2 * LANES,), jnp.float32),
            pltpu.SemaphoreType.DMA,
            pltpu.SemaphoreType.DMA,
            pltpu.VMEM((LANES,), jnp.float32),
            pltpu.VMEM((N_LAYERS * LANES,), jnp.float32),
            pltpu.VMEM((LANES,), jnp.float32),
            pltpu.VMEM_SHARED((N_LAYERS * LANES,), jnp.float32),
        ],
        compiler_params=pltpu.CompilerParams(needs_layout_passes=False),
    )(_select_body)


def _select_body(scores_hbm, aux_hbm, out_hbm,
                 scores_v, aux_v, sem1, sem2, row_v, prod_v, out_v, shared):
    c = lax.axis_index("c")
    sid = lax.axis_index("s")
    lane = lax.iota(jnp.int32, LANES)

    @pl.when((c == 0) & (sid < N_LAYERS))
    def _():
        layer = sid
        h1 = pltpu.async_copy(scores_hbm.at[layer], scores_v, sem1)
        h2 = pltpu.async_copy(aux_hbm, aux_v, sem2)
        h2.wait()
        h1.wait()
        mask = lane == layer
        s_vec = aux_v[pl.ds(0, LANES)]
        y_vec = aux_v[pl.ds(LANES, LANES)]
        t_vec = s_vec.astype(jnp.int32)
        k_vec = t_vec + jnp.where(t_vec.astype(jnp.float32) < s_vec, 1, 0)
        k_vec = jnp.minimum(jnp.maximum(k_vec, 0), D)
        k = jnp.sum(jnp.where(mask, k_vec, 0))
        y_i = jnp.sum(jnp.where(mask, y_vec, 0.0))

        def mmbody(jj, mnmx):
            mn, mx = mnmx
            for u in range(UNROLL):
                v = plsc.bitcast(
                    scores_v[pl.ds((jj * UNROLL + u) * LANES, LANES)],
                    jnp.int32)
                mn = jnp.minimum(mn, v)
                mx = jnp.maximum(mx, v)
            return (mn, mx)

        mn_v, mx_v = lax.fori_loop(
            0, CHUNKS // UNROLL, mmbody,
            (jnp.full((LANES,), MAX_FINITE_BITS, jnp.int32),
             jnp.zeros((LANES,), jnp.int32)))

        def count_le(mid):
            mid_vec = jnp.full((LANES,), mid, dtype=jnp.int32)

            def cbody(jj, cnt):
                for u in range(UNROLL):
                    v = plsc.bitcast(
                        scores_v[pl.ds((jj * UNROLL + u) * LANES, LANES)],
                        jnp.int32)
                    cnt = cnt + jnp.where(v <= mid_vec, 1, 0)
                return cnt

            cnt = lax.fori_loop(0, CHUNKS // UNROLL, cbody,
                                jnp.zeros((LANES,), jnp.int32))
            return jnp.sum(cnt)

        def bcond(lohi):
            lo, hi = lohi
            return lo < hi

        def bbody(lohi):
            lo, hi = lohi
            mid = lo + ((hi - lo) >> 1)
            ge = count_le(mid) >= k
            return (jnp.where(ge, lo, mid + 1), jnp.where(ge, mid, hi))

        lo, _ = lax.while_loop(
            bcond, bbody, (jnp.min(mn_v), jnp.max(mx_v)))
        t_bits = jnp.full((LANES,), lo, dtype=jnp.int32)

        def fbody(jj, carry):
            sm, cl = carry
            for u in range(UNROLL):
                sv = scores_v[pl.ds((jj * UNROLL + u) * LANES, LANES)]
                bv = plsc.bitcast(sv, jnp.int32)
                lt = bv < t_bits
                sm = sm + jnp.where(lt, sv, 0.0)
                cl = cl + jnp.where(lt, 1, 0)
            return (sm, cl)

        sm, cl = lax.fori_loop(
            0, CHUNKS // UNROLL, fbody,
            (jnp.zeros((LANES,), jnp.float32), jnp.zeros((LANES,), jnp.int32)))
        sum_lt = jnp.sum(sm)
        cnt_lt = jnp.sum(cl)
        rem_vec = jnp.full((LANES,), k - cnt_lt, dtype=jnp.int32)
        val_vec = (jnp.full((LANES,), sum_lt, dtype=jnp.float32)
                   + rem_vec.astype(jnp.float32)
                   * plsc.bitcast(t_bits, jnp.float32))
        row_v[...] = jnp.where(mask, y_vec * val_vec, 0.0)
        pltpu.sync_copy(row_v, shared.at[pl.ds(layer * LANES, LANES)])

    plsc.subcore_barrier()

    @pl.when((c == 0) & (sid == 0))
    def _():
        pltpu.sync_copy(shared, prod_v)

        def abody(j, acc):
            return acc + prod_v[pl.ds(j * LANES, LANES)]

        acc = lax.fori_loop(0, N_LAYERS, abody,
                            jnp.zeros((LANES,), jnp.float32))
        out_v[...] = jnp.full((LANES,), jnp.sum(acc), dtype=jnp.float32)
        pltpu.sync_copy(out_v, out_hbm)


def kernel(weights, s, y):
    scores = _scores_call(weights, weights).reshape(N_LAYERS, D)
    aux = jnp.zeros((2 * LANES,), jnp.float32)
    aux = aux.at[:N_LAYERS].set(s).at[LANES:LANES + N_LAYERS].set(y)
    out16 = _build_select_call()(scores, aux)
    return out16[0]
